# baseline (speedup 1.0000x reference)
import functools

import jax
import jax.numpy as jnp
from jax import lax
from jax.experimental import pallas as pl
from jax.experimental.pallas import tpu as pltpu
from jax.experimental.pallas import tpu_sc as plsc

NC = 2
NS = 16
NW = NC * NS
K = 128


def _mesh():
    return plsc.VectorSubcoreMesh(core_axis_name="c", subcore_axis_name="s")


def _row_padding(N):
    NP = -(-N // 8) * 8 + 8
    NPT = -(-(-(-NP // NS)) // 8) * 8
    LAST = NP - NPT * (NS - 1)
    assert 0 < LAST <= NPT and NP > N
    return NP, NPT, LAST


def _init_table(tab, zeros_hbm, s, NPT, LAST):
    if LAST == NPT:
        pltpu.sync_copy(zeros_hbm, tab.at[pl.ds(s * NPT, NPT)])
    else:
        @pl.when(s < NS - 1)
        def _():
            pltpu.sync_copy(zeros_hbm, tab.at[pl.ds(s * NPT, NPT)])

        @pl.when(s == NS - 1)
        def _():
            pltpu.sync_copy(zeros_hbm.at[pl.ds(0, LAST)],
                            tab.at[pl.ds((NS - 1) * NPT, LAST)])


def _read_table(tab, out_hbm, c, s, NPT, LAST):
    if LAST == NPT:
        pltpu.sync_copy(tab.at[pl.ds(s * NPT, NPT)],
                        out_hbm.at[c, pl.ds(s * NPT, NPT)])
    else:
        @pl.when(s < NS - 1)
        def _():
            pltpu.sync_copy(tab.at[pl.ds(s * NPT, NPT)],
                            out_hbm.at[c, pl.ds(s * NPT, NPT)])

        @pl.when(s == NS - 1)
        def _():
            pltpu.sync_copy(tab.at[pl.ds((NS - 1) * NPT, LAST)],
                            out_hbm.at[c, pl.ds((NS - 1) * NPT, LAST)])


def _make_deg_kernel(N, E, D):
    NRT = E // K
    NR = NRT // NW
    assert NRT == NR * NW and NR % 2 == 0 and NR >= 4
    NP, NPT, LAST = _row_padding(N)

    @functools.partial(
        pl.kernel,
        out_type=jax.ShapeDtypeStruct((NC, NP, D), jnp.float32),
        mesh=_mesh(),
        scratch_types=[
            pltpu.VMEM_SHARED((NP, D), jnp.float32),
            pltpu.VMEM((2, K), jnp.int32),
            pltpu.VMEM((2, K), jnp.int32),
            pltpu.VMEM((K, D), jnp.float32),
            pltpu.SemaphoreType.DMA,
            pltpu.SemaphoreType.DMA,
        ],
    )
    def deg_kernel(sd_hbm, ones_hbm, zeros_hbm, out_hbm, degsp, idxb0, idxb1,
                   ones_v, ssem0, ssem1):
        c = lax.axis_index("c")
        s = lax.axis_index("s")
        w = c * NS + s
        base = w * NR
        _init_table(degsp, zeros_hbm, s, NPT, LAST)
        pltpu.sync_copy(ones_hbm, ones_v)
        plsc.subcore_barrier()

        def load_idx(j, buf):
            pltpu.sync_copy(sd_hbm.at[base + j], buf)

        def fire(buf, sem):
            pltpu.async_copy(ones_v, degsp.at[buf.at[1]], sem, add=True)

        def drain(buf, sem):
            pltpu.make_async_copy(ones_v, degsp.at[buf.at[1]], sem).wait()

        load_idx(0, idxb0)
        load_idx(1, idxb1)
        fire(idxb0, ssem0)
        fire(idxb1, ssem1)

        def pair(p, carry):
            j0 = 2 * p
            drain(idxb0, ssem0)
            load_idx(j0 + 2, idxb0)
            fire(idxb0, ssem0)
            drain(idxb1, ssem1)
            load_idx(j0 + 3, idxb1)
            fire(idxb1, ssem1)
            return carry

        lax.fori_loop(0, NR // 2 - 1, pair, 0)
        drain(idxb0, ssem0)
        drain(idxb1, ssem1)
        plsc.subcore_barrier()
        _read_table(degsp, out_hbm, c, s, NPT, LAST)

    return deg_kernel


def _make_edge_kernel(N, E, D):
    NRT = E // K
    NR = NRT // NW
    assert NRT == NR * NW and NR % 2 == 0 and NR >= 4
    NP, NPT, LAST = _row_padding(N)

    @functools.partial(
        pl.kernel,
        out_type=jax.ShapeDtypeStruct((NC, NP, D), jnp.float32),
        mesh=_mesh(),
        scratch_types=[
            pltpu.VMEM_SHARED((NP, D), jnp.float32),
            pltpu.VMEM((2, K), jnp.int32),
            pltpu.VMEM((2, K), jnp.int32),
            pltpu.VMEM((K, D), jnp.float32),
            pltpu.VMEM((K, D), jnp.float32),
            pltpu.SemaphoreType.DMA,
            pltpu.SemaphoreType.DMA,
        ],
    )
    def edge_kernel(hs_hbm, sd_hbm, zeros_hbm, out_hbm, accsp,
                    idxb0, idxb1, rows0, rows1, gsem0, gsem1):
        c = lax.axis_index("c")
        s = lax.axis_index("s")
        w = c * NS + s
        base = w * NR
        _init_table(accsp, zeros_hbm, s, NPT, LAST)
        plsc.subcore_barrier()

        def load_idx(j, buf):
            pltpu.sync_copy(sd_hbm.at[base + j], buf)

        def fire_g(buf, rows, sem):
            pltpu.async_copy(hs_hbm.at[buf.at[0]], rows, sem)

        def drain_g(buf, rows, sem):
            pltpu.make_async_copy(hs_hbm.at[buf.at[0]], rows, sem).wait()

        def scat(buf, rows):
            pltpu.sync_copy(rows, accsp.at[buf.at[1]], add=True)

        load_idx(0, idxb0)
        load_idx(1, idxb1)
        fire_g(idxb0, rows0, gsem0)
        fire_g(idxb1, rows1, gsem1)

        def pair(p, carry):
            j0 = 2 * p
            drain_g(idxb0, rows0, gsem0)
            scat(idxb0, rows0)
            load_idx(j0 + 2, idxb0)
            fire_g(idxb0, rows0, gsem0)
            drain_g(idxb1, rows1, gsem1)
            scat(idxb1, rows1)
            load_idx(j0 + 3, idxb1)
            fire_g(idxb1, rows1, gsem1)
            return carry

        lax.fori_loop(0, NR // 2 - 1, pair, 0)
        drain_g(idxb0, rows0, gsem0)
        scat(idxb0, rows0)
        drain_g(idxb1, rows1, gsem1)
        scat(idxb1, rows1)
        plsc.subcore_barrier()
        _read_table(accsp, out_hbm, c, s, NPT, LAST)

    return edge_kernel


def _dis_from_deg(degp_ref):
    deg = degp_ref[0][:, 0:1] + degp_ref[1][:, 0:1] + 1.0
    return lax.rsqrt(deg)


def _tc_first(x_ref, w_ref, degp_ref, hs_ref):
    dis = _dis_from_deg(degp_ref)
    h = jnp.dot(x_ref[...], w_ref[...], preferred_element_type=jnp.float32,
                precision=lax.Precision.HIGHEST)
    hs_ref[...] = h * dis


def _tc_mid(acc_ref, hs_ref, degp_ref, b_ref, w_ref, out_ref):
    dis = _dis_from_deg(degp_ref)
    t = dis * (acc_ref[0] + acc_ref[1] + hs_ref[...]) + b_ref[...]
    o1 = jnp.maximum(t, 0.0)
    h2 = jnp.dot(o1, w_ref[...], preferred_element_type=jnp.float32,
                 precision=lax.Precision.HIGHEST)
    out_ref[...] = h2 * dis


def _tc_last(acc_ref, hs_ref, degp_ref, b_ref, out_ref):
    dis = _dis_from_deg(degp_ref)
    out_ref[...] = dis * (acc_ref[0] + acc_ref[1] + hs_ref[...]) + b_ref[...]


def kernel(x, edge_index, W1, b1, W2, b2):
    N, D = x.shape
    E = edge_index.shape[1]
    assert N % NS == 0

    blk = NW * K * 2
    EP = -(-E // blk) * blk
    idt = edge_index.dtype
    if EP != E:
        src_pad = jnp.concatenate([edge_index[0], jnp.zeros((EP - E,), idt)])
        dst_pad = jnp.concatenate([edge_index[1], jnp.full((EP - E,), N, idt)])
    else:
        src_pad, dst_pad = edge_index[0], edge_index[1]
    sd = jnp.stack([src_pad.reshape(EP // K, K),
                    dst_pad.reshape(EP // K, K)], axis=1)
    f32 = jnp.float32
    NP, NPT, _ = _row_padding(N)
    onesD = jnp.ones((K, D), f32)
    zerosD = jnp.zeros((NPT, D), f32)

    degp = _make_deg_kernel(N, EP, D)(sd, onesD, zerosD)[:, :N]

    BN = 400
    grid = (N // BN,)
    blk_nd = pl.BlockSpec((BN, D), lambda i: (i, 0))
    blk_w = pl.BlockSpec((D, D), lambda i: (0, 0))
    blk_deg = pl.BlockSpec((NC, BN, D), lambda i: (0, i, 0))
    blk_acc = pl.BlockSpec((NC, BN, D), lambda i: (0, i, 0))
    blk_b = pl.BlockSpec((1, D), lambda i: (0, 0))
    out_nd = jax.ShapeDtypeStruct((N, D), f32)

    hs1 = pl.pallas_call(
        _tc_first, grid=grid,
        in_specs=[blk_nd, blk_w, blk_deg],
        out_specs=blk_nd, out_shape=out_nd,
    )(x, W1, degp)

    edge_k = _make_edge_kernel(N, EP, D)
    acc1 = edge_k(hs1, sd, zerosD)[:, :N]

    hs2 = pl.pallas_call(
        _tc_mid, grid=grid,
        in_specs=[blk_acc, blk_nd, blk_deg, blk_b, blk_w],
        out_specs=blk_nd, out_shape=out_nd,
    )(acc1, hs1, degp, b1.reshape(1, D), W2)

    acc2 = edge_k(hs2, sd, zerosD)[:, :N]

    out = pl.pallas_call(
        _tc_last, grid=grid,
        in_specs=[blk_acc, blk_nd, blk_deg, blk_b],
        out_specs=blk_nd, out_shape=out_nd,
    )(acc2, hs2, degp, b2.reshape(1, D))
    return out


# speedup vs baseline: 2.7032x; 2.7032x over previous
import functools

import jax
import jax.numpy as jnp
from jax import lax
from jax.experimental import pallas as pl
from jax.experimental.pallas import tpu as pltpu
from jax.experimental.pallas import tpu_sc as plsc

NC = 2
NS = 16
NW = NC * NS
K = 128


def _mesh():
    return plsc.VectorSubcoreMesh(core_axis_name="c", subcore_axis_name="s")


def _row_padding(N):
    NP = -(-N // 8) * 8 + 8
    NPT = -(-(-(-NP // NS)) // 8) * 8
    LAST = NP - NPT * (NS - 1)
    assert 0 < LAST <= NPT and NP > N
    return NP, NPT, LAST


def _init_table(tab, zeros_hbm, s, NPT, LAST):
    if LAST == NPT:
        pltpu.sync_copy(zeros_hbm, tab.at[pl.ds(s * NPT, NPT)])
    else:
        @pl.when(s < NS - 1)
        def _():
            pltpu.sync_copy(zeros_hbm, tab.at[pl.ds(s * NPT, NPT)])

        @pl.when(s == NS - 1)
        def _():
            pltpu.sync_copy(zeros_hbm.at[pl.ds(0, LAST)],
                            tab.at[pl.ds((NS - 1) * NPT, LAST)])


def _read_table(tab, out_hbm, c, s, NPT, LAST):
    if LAST == NPT:
        pltpu.sync_copy(tab.at[pl.ds(s * NPT, NPT)],
                        out_hbm.at[c, pl.ds(s * NPT, NPT)])
    else:
        @pl.when(s < NS - 1)
        def _():
            pltpu.sync_copy(tab.at[pl.ds(s * NPT, NPT)],
                            out_hbm.at[c, pl.ds(s * NPT, NPT)])

        @pl.when(s == NS - 1)
        def _():
            pltpu.sync_copy(tab.at[pl.ds((NS - 1) * NPT, LAST)],
                            out_hbm.at[c, pl.ds((NS - 1) * NPT, LAST)])


def _make_deg_kernel(N, E, D):
    NRT = E // K
    NR = NRT // NW
    assert NRT == NR * NW and NR % 2 == 0 and NR >= 4
    NP, NPT, LAST = _row_padding(N)

    @functools.partial(
        pl.kernel,
        out_type=jax.ShapeDtypeStruct((NC, NP, D), jnp.float32),
        mesh=_mesh(),
        scratch_types=[
            pltpu.VMEM_SHARED((NP, D), jnp.float32),
            pltpu.VMEM((2, K), jnp.int32),
            pltpu.VMEM((2, K), jnp.int32),
            pltpu.VMEM((K, D), jnp.float32),
            pltpu.SemaphoreType.DMA,
            pltpu.SemaphoreType.DMA,
        ],
    )
    def deg_kernel(sd_hbm, ones_hbm, zeros_hbm, out_hbm, degsp, idxb0, idxb1,
                   ones_v, ssem0, ssem1):
        c = lax.axis_index("c")
        s = lax.axis_index("s")
        w = c * NS + s
        base = w * NR
        _init_table(degsp, zeros_hbm, s, NPT, LAST)
        pltpu.sync_copy(ones_hbm, ones_v)
        plsc.subcore_barrier()

        def load_idx(j, buf):
            pltpu.sync_copy(sd_hbm.at[base + j], buf)

        def fire(buf, sem):
            pltpu.async_copy(ones_v, degsp.at[buf.at[1]], sem, add=True)

        def drain(buf, sem):
            pltpu.make_async_copy(ones_v, degsp.at[buf.at[1]], sem).wait()

        load_idx(0, idxb0)
        load_idx(1, idxb1)
        fire(idxb0, ssem0)
        fire(idxb1, ssem1)

        def pair(p, carry):
            j0 = 2 * p
            drain(idxb0, ssem0)
            load_idx(j0 + 2, idxb0)
            fire(idxb0, ssem0)
            drain(idxb1, ssem1)
            load_idx(j0 + 3, idxb1)
            fire(idxb1, ssem1)
            return carry

        lax.fori_loop(0, NR // 2 - 1, pair, 0)
        drain(idxb0, ssem0)
        drain(idxb1, ssem1)
        plsc.subcore_barrier()
        _read_table(degsp, out_hbm, c, s, NPT, LAST)

    return deg_kernel


def _make_edge_kernel(N, E, D):
    NRT = E // K
    NR = NRT // NW
    assert NRT == NR * NW and NR % 2 == 0 and NR >= 4
    NP, NPT, LAST = _row_padding(N)

    @functools.partial(
        pl.kernel,
        out_type=jax.ShapeDtypeStruct((NC, NP, D), jnp.float32),
        mesh=_mesh(),
        scratch_types=[
            pltpu.VMEM_SHARED((NP, D), jnp.float32),
            pltpu.VMEM((2, K), jnp.int32),
            pltpu.VMEM((2, K), jnp.int32),
            pltpu.VMEM((K, D), jnp.float32),
            pltpu.VMEM((K, D), jnp.float32),
            pltpu.SemaphoreType.DMA,
            pltpu.SemaphoreType.DMA,
        ],
    )
    def edge_kernel(hs_hbm, sd_hbm, zeros_hbm, out_hbm, accsp,
                    idxb0, idxb1, rows0, rows1, gsem0, gsem1):
        c = lax.axis_index("c")
        s = lax.axis_index("s")
        w = c * NS + s
        base = w * NR
        _init_table(accsp, zeros_hbm, s, NPT, LAST)
        plsc.subcore_barrier()

        def load_idx(j, buf):
            pltpu.sync_copy(sd_hbm.at[base + j], buf)

        def fire_g(buf, rows, sem):
            pltpu.async_copy(hs_hbm.at[buf.at[0]], rows, sem)

        def drain_g(buf, rows, sem):
            pltpu.make_async_copy(hs_hbm.at[buf.at[0]], rows, sem).wait()

        def scat(buf, rows):
            pltpu.sync_copy(rows, accsp.at[buf.at[1]], add=True)

        load_idx(0, idxb0)
        load_idx(1, idxb1)
        fire_g(idxb0, rows0, gsem0)
        fire_g(idxb1, rows1, gsem1)

        def pair(p, carry):
            j0 = 2 * p
            drain_g(idxb0, rows0, gsem0)
            scat(idxb0, rows0)
            load_idx(j0 + 2, idxb0)
            fire_g(idxb0, rows0, gsem0)
            drain_g(idxb1, rows1, gsem1)
            scat(idxb1, rows1)
            load_idx(j0 + 3, idxb1)
            fire_g(idxb1, rows1, gsem1)
            return carry

        lax.fori_loop(0, NR // 2 - 1, pair, 0)
        drain_g(idxb0, rows0, gsem0)
        scat(idxb0, rows0)
        drain_g(idxb1, rows1, gsem1)
        scat(idxb1, rows1)
        plsc.subcore_barrier()
        _read_table(accsp, out_hbm, c, s, NPT, LAST)

    return edge_kernel


def _dis_from_deg(degp_ref):
    deg = degp_ref[0][:, 0:1] + degp_ref[1][:, 0:1] + 1.0
    return lax.rsqrt(deg)


def _tc_first(x_ref, w_ref, degp_ref, hs_ref):
    dis = _dis_from_deg(degp_ref)
    h = jnp.dot(x_ref[...], w_ref[...], preferred_element_type=jnp.float32,
                precision=lax.Precision.HIGHEST)
    hs_ref[...] = h * dis


def _tc_mid(acc_ref, hs_ref, degp_ref, b_ref, w_ref, out_ref):
    dis = _dis_from_deg(degp_ref)
    t = dis * (acc_ref[0] + acc_ref[1] + hs_ref[...]) + b_ref[...]
    o1 = jnp.maximum(t, 0.0)
    h2 = jnp.dot(o1, w_ref[...], preferred_element_type=jnp.float32,
                 precision=lax.Precision.HIGHEST)
    out_ref[...] = h2 * dis


def _tc_last(acc_ref, hs_ref, degp_ref, b_ref, out_ref):
    dis = _dis_from_deg(degp_ref)
    out_ref[...] = dis * (acc_ref[0] + acc_ref[1] + hs_ref[...]) + b_ref[...]


def kernel(x, edge_index, W1, b1, W2, b2):
    N, D = x.shape
    E = edge_index.shape[1]
    assert N % NS == 0

    blk = NW * K * 2
    EP = -(-E // blk) * blk
    idt = edge_index.dtype
    if EP != E:
        fill = jnp.arange(EP - E, dtype=idt) % N
        src_pad = jnp.concatenate([edge_index[0], fill])
        dst_pad = jnp.concatenate([edge_index[1], jnp.full((EP - E,), N, idt)])
    else:
        src_pad, dst_pad = edge_index[0], edge_index[1]
    sd = jnp.stack([src_pad.reshape(EP // K, K),
                    dst_pad.reshape(EP // K, K)], axis=1)
    f32 = jnp.float32
    NP, NPT, _ = _row_padding(N)
    onesD = jnp.ones((K, D), f32)
    zerosD = jnp.zeros((NPT, D), f32)

    degp = _make_deg_kernel(N, EP, D)(sd, onesD, zerosD)[:, :N]

    BN = 400
    grid = (N // BN,)
    blk_nd = pl.BlockSpec((BN, D), lambda i: (i, 0))
    blk_w = pl.BlockSpec((D, D), lambda i: (0, 0))
    blk_deg = pl.BlockSpec((NC, BN, D), lambda i: (0, i, 0))
    blk_acc = pl.BlockSpec((NC, BN, D), lambda i: (0, i, 0))
    blk_b = pl.BlockSpec((1, D), lambda i: (0, 0))
    out_nd = jax.ShapeDtypeStruct((N, D), f32)

    hs1 = pl.pallas_call(
        _tc_first, grid=grid,
        in_specs=[blk_nd, blk_w, blk_deg],
        out_specs=blk_nd, out_shape=out_nd,
    )(x, W1, degp)

    edge_k = _make_edge_kernel(N, EP, D)
    acc1 = edge_k(hs1, sd, zerosD)[:, :N]

    hs2 = pl.pallas_call(
        _tc_mid, grid=grid,
        in_specs=[blk_acc, blk_nd, blk_deg, blk_b, blk_w],
        out_specs=blk_nd, out_shape=out_nd,
    )(acc1, hs1, degp, b1.reshape(1, D), W2)

    acc2 = edge_k(hs2, sd, zerosD)[:, :N]

    out = pl.pallas_call(
        _tc_last, grid=grid,
        in_specs=[blk_acc, blk_nd, blk_deg, blk_b],
        out_specs=blk_nd, out_shape=out_nd,
    )(acc2, hs2, degp, b2.reshape(1, D))
    return out
